# Initial kernel scaffold; baseline (speedup 1.0000x reference)
import jax
import jax.numpy as jnp
from jax.experimental import pallas as pl


def kernel(logits_0, logits_1, logits_2, targets):
    raise NotImplementedError("write your pallas kernel here")


# speedup vs baseline: 4.5172x; 4.5172x over previous
import functools

import jax
import jax.numpy as jnp
from jax.experimental import pallas as pl
from jax.experimental.pallas import tpu as pltpu

PCTS = (0.5, 0.75, 1.0)
PAD = 0


def _stats_body(t_ref, l0_ref, l1_ref, l2_ref,
                conf_ref, ce_ref, corr_ref):
    tgt = t_ref[0, 0, :]
    tb, d = l0_ref.shape
    tgt_col = tgt.reshape(tb, 1)
    lane = jax.lax.broadcasted_iota(jnp.int32, (tb, d), 1)
    for lvl, ref in enumerate((l0_ref, l1_ref, l2_ref)):
        x = ref[...]
        m = jnp.max(x, axis=1, keepdims=True)
        pred = jnp.min(jnp.where(x == m, lane, d), axis=1)
        ls = jnp.log(jnp.sum(jnp.exp(x - m), axis=1, keepdims=True))
        xt = jnp.sum(jnp.where(lane == tgt_col, x, 0.0), axis=1)
        conf_ref[lvl, 0, 0, :] = -ls[:, 0]
        ce_ref[lvl, 0, 0, :] = ls[:, 0] + m[:, 0] - xt
        corr_ref[lvl, 0, 0, :] = (pred == tgt).astype(jnp.int32)


def _select_body(t_ref, conf_ref, ce_ref, corr_ref, out_ref, *, n_levels):
    B, T = t_ref.shape
    C = 512
    idx_row = jax.lax.broadcasted_iota(jnp.int32, (1, T), 1)
    total_loss = jnp.float32(0.0)
    total_tokens = jnp.float32(0.0)
    for b in range(B):
        tgt = t_ref[b, :].reshape(1, T)
        valid = tgt != PAD
        num_valid = jnp.sum(valid.astype(jnp.float32))
        sel = jnp.zeros((1, T), dtype=jnp.bool_)
        for lvl in range(n_levels):
            conf = conf_ref[lvl, b, :].reshape(1, T)
            ce = ce_ref[lvl, b, :].reshape(1, T)
            corr = corr_ref[lvl, b, :].reshape(1, T) != 0
            correct_mask = corr & valid & (~sel)
            sel = sel | correct_mask
            n_lvl = jnp.ceil(num_valid * PCTS[lvl])
            num_sel = jnp.sum((sel & valid).astype(jnp.float32))
            need = jnp.maximum(n_lvl - num_sel, 0.0)
            rem = valid & (~sel)
            num_rem = jnp.sum(rem.astype(jnp.float32))
            k_sel = jnp.minimum(need, num_rem)
            confm = jnp.where(rem, conf, -jnp.inf)
            rank = jnp.zeros((1, T), dtype=jnp.float32)
            for c in range(0, T, C):
                ck = confm[0, c:c + C].reshape(C, 1)
                kk = idx_row[0, c:c + C].reshape(C, 1)
                beats = (ck > confm) | ((ck == confm) & (kk < idx_row))
                rank = rank + jnp.sum(
                    beats.astype(jnp.float32), axis=0, keepdims=True)
            add = rem & (rank < k_sel)
            sel = sel | add
            new_sel = correct_mask | add
            nsf = new_sel.astype(jnp.float32)
            total_loss = total_loss + jnp.sum(nsf * ce)
            total_tokens = total_tokens + jnp.sum(nsf)
    final = jnp.where(
        total_tokens == 0.0, 0.0,
        total_loss / jnp.maximum(total_tokens, 1.0))
    out_ref[...] = jnp.broadcast_to(final, (1, 1))


@jax.jit
def kernel(logits_0, logits_1, logits_2, targets):
    B, T, D = logits_0.shape
    TB = 256
    n_blk = (B * T) // TB
    tgt32 = targets.astype(jnp.int32)
    tgt_blk = tgt32.reshape(n_blk, 1, TB)
    flat = [x.reshape(B * T, D) for x in (logits_0, logits_1, logits_2)]

    stats_out = [
        jax.ShapeDtypeStruct((3, n_blk, 1, TB), jnp.float32),
        jax.ShapeDtypeStruct((3, n_blk, 1, TB), jnp.float32),
        jax.ShapeDtypeStruct((3, n_blk, 1, TB), jnp.int32),
    ]

    def stats_wrap(t_ref, l0, l1, l2, conf, ce, corr):
        _stats_body(t_ref, l0, l1, l2, conf, ce, corr)

    conf, ce, corr = pl.pallas_call(
        stats_wrap,
        grid=(n_blk,),
        in_specs=[
            pl.BlockSpec((1, 1, TB), lambda i: (i, 0, 0)),
            pl.BlockSpec((TB, D), lambda i: (i, 0)),
            pl.BlockSpec((TB, D), lambda i: (i, 0)),
            pl.BlockSpec((TB, D), lambda i: (i, 0)),
        ],
        out_specs=[
            pl.BlockSpec((3, 1, 1, TB), lambda i: (0, i, 0, 0)),
            pl.BlockSpec((3, 1, 1, TB), lambda i: (0, i, 0, 0)),
            pl.BlockSpec((3, 1, 1, TB), lambda i: (0, i, 0, 0)),
        ],
        out_shape=stats_out,
    )(tgt_blk, *flat)

    conf = conf.reshape(3, B, T)
    ce = ce.reshape(3, B, T)
    corr = corr.reshape(3, B, T)

    loss = pl.pallas_call(
        functools.partial(_select_body, n_levels=3),
        out_shape=jax.ShapeDtypeStruct((1, 1), jnp.float32),
    )(tgt32, conf, ce, corr)
    return loss[0, 0]


# speedup vs baseline: 5.5247x; 1.2230x over previous
import functools

import jax
import jax.numpy as jnp
from jax.experimental import pallas as pl
from jax.experimental.pallas import tpu as pltpu

PCTS = (0.5, 0.75, 1.0)
PAD = 0


def _stats_body(t_ref, l0_ref, l1_ref, l2_ref,
                conf_ref, ce_ref, corr_ref):
    tgt = t_ref[0, 0, :]
    tb, d = l0_ref.shape
    tgt_col = tgt.reshape(tb, 1)
    lane = jax.lax.broadcasted_iota(jnp.int32, (tb, d), 1)
    for lvl, ref in enumerate((l0_ref, l1_ref, l2_ref)):
        x = ref[...]
        m = jnp.max(x, axis=1, keepdims=True)
        pred = jnp.min(jnp.where(x == m, lane, d), axis=1)
        ls = jnp.log(jnp.sum(jnp.exp(x - m), axis=1, keepdims=True))
        xt = jnp.sum(jnp.where(lane == tgt_col, x, 0.0), axis=1)
        conf_ref[lvl, 0, 0, :] = -ls[:, 0]
        ce_ref[lvl, 0, 0, :] = ls[:, 0] + m[:, 0] - xt
        corr_ref[lvl, 0, 0, :] = (pred == tgt).astype(jnp.int32)


def _select_body(t_ref, conf_ref, ce_ref, corr_ref, out_ref, *, n_levels):
    B, T = t_ref.shape
    MIN32 = jnp.int32(-2**31)
    n_idx_bits = max(1, (T - 1).bit_length())
    idx_row = jax.lax.broadcasted_iota(jnp.int32, (B, T), 1)
    tgt = t_ref[...]
    valid = tgt != PAD
    num_valid = jnp.sum(valid.astype(jnp.float32), axis=1, keepdims=True)
    sel = jnp.zeros((B, T), dtype=jnp.bool_)
    total_loss = jnp.float32(0.0)
    total_tokens = jnp.float32(0.0)
    for lvl in range(n_levels):
        conf = conf_ref[lvl, :, :]
        ce = ce_ref[lvl, :, :]
        corr = corr_ref[lvl, :, :] != 0
        correct_mask = corr & valid & (~sel)
        sel = sel | correct_mask
        n_lvl = jnp.ceil(num_valid * PCTS[lvl])
        num_sel = jnp.sum((sel & valid).astype(jnp.float32),
                          axis=1, keepdims=True)
        need = jnp.maximum(n_lvl - num_sel, 0.0)
        rem = valid & (~sel)
        num_rem = jnp.sum(rem.astype(jnp.float32), axis=1, keepdims=True)
        k_sel = jnp.minimum(need, num_rem)
        confz = jnp.where(conf == 0.0, 0.0, conf)
        confm = jnp.where(rem, confz, -jnp.inf)
        fb = jax.lax.bitcast_convert_type(confm, jnp.int32)
        skey = jnp.where(fb >= 0, fb, ~(fb ^ MIN32))
        c_u = jnp.zeros((B, 1), dtype=jnp.int32)
        for bit in range(31, -1, -1):
            cand = c_u | (jnp.int32(1) << bit)
            scand = cand ^ MIN32
            cnt = jnp.sum((skey >= scand).astype(jnp.float32),
                          axis=1, keepdims=True)
            c_u = jnp.where(cnt >= k_sel, cand, c_u)
        s_star = c_u ^ MIN32
        gt = skey > s_star
        cnt_gt = jnp.sum(gt.astype(jnp.float32), axis=1, keepdims=True)
        eq = skey == s_star
        r = k_sel - cnt_gt
        m_cut = jnp.zeros((B, 1), dtype=jnp.int32)
        for bit in range(n_idx_bits - 1, -1, -1):
            cand = m_cut | (jnp.int32(1) << bit)
            f_cnt = jnp.sum((eq & (idx_row < cand)).astype(jnp.float32),
                            axis=1, keepdims=True)
            m_cut = jnp.where(f_cnt < r, cand, m_cut)
        add = gt | (eq & (idx_row <= m_cut))
        sel = sel | add
        new_sel = correct_mask | add
        nsf = new_sel.astype(jnp.float32)
        total_loss = total_loss + jnp.sum(nsf * ce)
        total_tokens = total_tokens + jnp.sum(nsf)
    final = jnp.where(
        total_tokens == 0.0, 0.0,
        total_loss / jnp.maximum(total_tokens, 1.0))
    out_ref[...] = jnp.broadcast_to(final, (1, 1))


@jax.jit
def kernel(logits_0, logits_1, logits_2, targets):
    B, T, D = logits_0.shape
    TB = 256
    n_blk = (B * T) // TB
    tgt32 = targets.astype(jnp.int32)
    tgt_blk = tgt32.reshape(n_blk, 1, TB)
    flat = [x.reshape(B * T, D) for x in (logits_0, logits_1, logits_2)]

    stats_out = [
        jax.ShapeDtypeStruct((3, n_blk, 1, TB), jnp.float32),
        jax.ShapeDtypeStruct((3, n_blk, 1, TB), jnp.float32),
        jax.ShapeDtypeStruct((3, n_blk, 1, TB), jnp.int32),
    ]

    def stats_wrap(t_ref, l0, l1, l2, conf, ce, corr):
        _stats_body(t_ref, l0, l1, l2, conf, ce, corr)

    conf, ce, corr = pl.pallas_call(
        stats_wrap,
        grid=(n_blk,),
        in_specs=[
            pl.BlockSpec((1, 1, TB), lambda i: (i, 0, 0)),
            pl.BlockSpec((TB, D), lambda i: (i, 0)),
            pl.BlockSpec((TB, D), lambda i: (i, 0)),
            pl.BlockSpec((TB, D), lambda i: (i, 0)),
        ],
        out_specs=[
            pl.BlockSpec((3, 1, 1, TB), lambda i: (0, i, 0, 0)),
            pl.BlockSpec((3, 1, 1, TB), lambda i: (0, i, 0, 0)),
            pl.BlockSpec((3, 1, 1, TB), lambda i: (0, i, 0, 0)),
        ],
        out_shape=stats_out,
    )(tgt_blk, *flat)

    conf = conf.reshape(3, B, T)
    ce = ce.reshape(3, B, T)
    corr = corr.reshape(3, B, T)

    loss = pl.pallas_call(
        functools.partial(_select_body, n_levels=3),
        out_shape=jax.ShapeDtypeStruct((1, 1), jnp.float32),
    )(tgt32, conf, ce, corr)
    return loss[0, 0]
